# Initial kernel scaffold; baseline (speedup 1.0000x reference)
import jax
import jax.numpy as jnp
from jax.experimental import pallas as pl


def kernel(x, cluster_centers):
    raise NotImplementedError("write your pallas kernel here")


# speedup vs baseline: 1.4229x; 1.4229x over previous
import jax
import jax.numpy as jnp
from jax.experimental import pallas as pl

NUM_CLUSTERS = 64
D_MODEL = 768
N_POINTS = 2048
TOP_K = 10

OUT_ROWS = N_POINTS * TOP_K
GATHER_BLOCK = 2048
GATHER_GRID = OUT_ROWS // GATHER_BLOCK


def _gather_body(idx_ref, x64_ref, out_ref):
    idx = idx_ref[:, :]
    lanes = jax.lax.broadcasted_iota(jnp.int32, (GATHER_BLOCK, NUM_CLUSTERS), 1)
    onehot = (idx == lanes).astype(jnp.float32)
    out_ref[:, :] = jnp.dot(onehot, x64_ref[:, :],
                            preferred_element_type=jnp.float32,
                            precision=jax.lax.Precision.HIGHEST)


def kernel(x, cluster_centers):
    diff = x[:, None, :] - cluster_centers[None, :, :]
    distances = jnp.linalg.norm(diff, axis=-1)
    _, indices = jax.lax.top_k(-distances, k=TOP_K)

    idx_flat = jnp.reshape(indices, (OUT_ROWS, 1)).astype(jnp.int32)
    x64 = x[:NUM_CLUSTERS]

    out = pl.pallas_call(
        _gather_body,
        grid=(GATHER_GRID,),
        in_specs=[
            pl.BlockSpec((GATHER_BLOCK, 1), lambda i: (i, 0)),
            pl.BlockSpec((NUM_CLUSTERS, D_MODEL), lambda i: (0, 0)),
        ],
        out_specs=pl.BlockSpec((GATHER_BLOCK, D_MODEL), lambda i: (i, 0)),
        out_shape=jax.ShapeDtypeStruct((OUT_ROWS, D_MODEL), jnp.float32),
    )(idx_flat, x64)
    return jnp.reshape(out, (1, OUT_ROWS, D_MODEL))
